# Initial kernel scaffold; baseline (speedup 1.0000x reference)
import jax
import jax.numpy as jnp
from jax.experimental import pallas as pl


def kernel(input_ids, position_ids, word_embeddings):
    raise NotImplementedError("write your pallas kernel here")


# speedup vs baseline: 1.0950x; 1.0950x over previous
import functools

import jax
import jax.numpy as jnp
from jax import lax
from jax.experimental import pallas as pl
from jax.experimental.pallas import tpu as pltpu
from jax.experimental.pallas import tpu_sc as plsc

NC = 2
NS = 16
NW = NC * NS

G = 128
GROUPS = 8
CHUNK = G * GROUPS


def _gather_kernel(hidden, n_chunks, idx_hbm, table_hbm, out_hbm,
                   idx_v, rows_v, sem):
    wid = lax.axis_index("s") * NC + lax.axis_index("c")
    group_base = wid * (n_chunks * GROUPS)
    row_base = wid * (n_chunks * CHUNK)

    def body(c, _):
        pltpu.sync_copy(idx_hbm.at[pl.ds(group_base + c * GROUPS, GROUPS)],
                        idx_v)
        copies = [
            pltpu.async_copy(table_hbm.at[idx_v.at[j]],
                             rows_v.at[pl.ds(j * G, G)], sem)
            for j in range(GROUPS)
        ]
        for cp in copies:
            cp.wait()
        pltpu.sync_copy(rows_v, out_hbm.at[pl.ds(row_base + c * CHUNK, CHUNK)])
        return ()

    lax.fori_loop(0, n_chunks, body, (), unroll=False)


def kernel(input_ids, position_ids, word_embeddings):
    batch, seq = input_ids.shape
    vocab, hidden = word_embeddings.shape
    n = batch * seq
    assert n % (NW * CHUNK) == 0
    n_chunks = n // (NW * CHUNK)

    idx_flat = input_ids.reshape(n // G, G)

    mesh = plsc.VectorSubcoreMesh(core_axis_name="c", subcore_axis_name="s")
    gather = pl.kernel(
        functools.partial(_gather_kernel, hidden, n_chunks),
        out_type=jax.ShapeDtypeStruct((n, hidden), jnp.float32),
        mesh=mesh,
        scratch_types=[
            pltpu.VMEM((GROUPS, G), jnp.int32),
            pltpu.VMEM((CHUNK, hidden), jnp.float32),
            pltpu.SemaphoreType.DMA,
        ],
        compiler_params=pltpu.CompilerParams(use_tc_tiling_on_sc=False),
    )
    out = gather(idx_flat, word_embeddings)
    return (out.reshape(batch, seq, hidden), position_ids)


# speedup vs baseline: 1.1084x; 1.0123x over previous
import functools

import jax
import jax.numpy as jnp
from jax import lax
from jax.experimental import pallas as pl
from jax.experimental.pallas import tpu as pltpu
from jax.experimental.pallas import tpu_sc as plsc

NC = 2
NS = 16
NW = NC * NS

G = 128
GROUPS = 10
CHUNK = G * GROUPS


def _gather_kernel(hidden, n_pairs, idx_hbm, table_hbm, out_hbm,
                   idx0, idx1, rows0, rows1, gsem, wsem0, wsem1):
    wid = lax.axis_index("s") * NC + lax.axis_index("c")
    n_chunks = 2 * n_pairs
    group_base = wid * (n_chunks * GROUPS)
    row_base = wid * (n_chunks * CHUNK)

    bufs = ((idx0, rows0, wsem0), (idx1, rows1, wsem1))

    def do_chunk(k, sub):
        idx_v, rows_v, wsem = bufs[sub]
        c = 2 * k + sub

        @pl.when(k > 0)
        def _():
            pltpu.make_async_copy(
                rows_v, out_hbm.at[pl.ds(row_base, CHUNK)], wsem).wait()

        pltpu.sync_copy(idx_hbm.at[pl.ds(group_base + c * GROUPS, GROUPS)],
                        idx_v)
        copies = [
            pltpu.async_copy(table_hbm.at[idx_v.at[j]],
                             rows_v.at[pl.ds(j * G, G)], gsem)
            for j in range(GROUPS)
        ]
        for cp in copies:
            cp.wait()
        pltpu.async_copy(rows_v, out_hbm.at[pl.ds(row_base + c * CHUNK, CHUNK)],
                         wsem)

    def body(k, _):
        do_chunk(k, 0)
        do_chunk(k, 1)
        return ()

    lax.fori_loop(0, n_pairs, body, (), unroll=False)

    for _, rows_v, wsem in bufs:
        pltpu.make_async_copy(
            rows_v, out_hbm.at[pl.ds(row_base, CHUNK)], wsem).wait()


def kernel(input_ids, position_ids, word_embeddings):
    batch, seq = input_ids.shape
    vocab, hidden = word_embeddings.shape
    n = batch * seq
    assert n % (NW * 2 * CHUNK) == 0
    n_pairs = n // (NW * 2 * CHUNK)

    idx_flat = input_ids.reshape(n // G, G)

    mesh = plsc.VectorSubcoreMesh(core_axis_name="c", subcore_axis_name="s")
    gather = pl.kernel(
        functools.partial(_gather_kernel, hidden, n_pairs),
        out_type=jax.ShapeDtypeStruct((n, hidden), jnp.float32),
        mesh=mesh,
        scratch_types=[
            pltpu.VMEM((GROUPS, G), jnp.int32),
            pltpu.VMEM((GROUPS, G), jnp.int32),
            pltpu.VMEM((CHUNK, hidden), jnp.float32),
            pltpu.VMEM((CHUNK, hidden), jnp.float32),
            pltpu.SemaphoreType.DMA,
            pltpu.SemaphoreType.DMA,
            pltpu.SemaphoreType.DMA,
        ],
        compiler_params=pltpu.CompilerParams(use_tc_tiling_on_sc=False),
    )
    out = gather(idx_flat, word_embeddings)
    return (out.reshape(batch, seq, hidden), position_ids)
